# Initial kernel scaffold; baseline (speedup 1.0000x reference)
import jax
import jax.numpy as jnp
from jax.experimental import pallas as pl


def kernel(x, edge_index, batch, W1s, b1s, W2s, b2s, gammas, betas):
    raise NotImplementedError("write your pallas kernel here")


# speedup vs baseline: 3.1941x; 3.1941x over previous
import functools

import jax
import jax.numpy as jnp
from jax import lax
from jax.experimental import pallas as pl
from jax.experimental.pallas import tpu as pltpu
from jax.experimental.pallas import tpu_sc as plsc

N = 10000
E = 160000
D = 256
HALF = 128
HID = 512
L = 3

NC = 2
NS = 16
K = 128

E_PAD = ((E + NS * K - 1) // (NS * K)) * (NS * K)
EPT = E_PAD // NS
CH = EPT // K
RPT = N // NS
N_ACC = N + 16


def _sc_body(h3_hbm, h2_hbm, src_hbm, dst3_hbm, out_hbm,
             sidx, didx, gidx, rows, sem, acc):
    c = lax.axis_index("c")
    s = lax.axis_index("s")
    r0 = s * RPT
    pltpu.sync_copy(h3_hbm.at[s, :, pl.ds(c * HALF, HALF)],
                    acc.at[pl.ds(r0, RPT)])
    pltpu.sync_copy(src_hbm.at[pl.ds(s * EPT, EPT)], sidx)
    pltpu.sync_copy(dst3_hbm.at[s], didx)
    plsc.subcore_barrier()

    def chunk(j, carry):
        for k in range(K // 16):
            v = sidx[pl.ds(j * K + 16 * k, 16)]
            gidx[pl.ds(16 * k, 16)] = v * 2 + c
        pltpu.async_copy(h2_hbm.at[gidx], rows, sem).wait()
        pltpu.sync_copy(rows, acc.at[didx.at[j]], add=True)
        return carry

    lax.fori_loop(0, CH, chunk, 0)
    plsc.subcore_barrier()
    pltpu.sync_copy(acc.at[pl.ds(r0, RPT)], out_hbm.at[c, s])


def _make_sc_segsum():
    mesh = plsc.VectorSubcoreMesh(core_axis_name="c", subcore_axis_name="s")
    return functools.partial(
        pl.kernel,
        mesh=mesh,
        out_type=jax.ShapeDtypeStruct((NC, NS, RPT, HALF), jnp.float32),
        scratch_types=[
            pltpu.VMEM((EPT,), jnp.int32),
            pltpu.VMEM((CH, K), jnp.int32),
            pltpu.VMEM((K,), jnp.int32),
            pltpu.VMEM((K, HALF), jnp.float32),
            pltpu.SemaphoreType.DMA,
            pltpu.VMEM_SHARED((N_ACC, HALF), jnp.float32),
        ],
    )(_sc_body)


_sc_segsum = _make_sc_segsum()


_BN_ROWS = 400


def _mlp_body(agg_ref, w1_ref, b1_ref, w2_ref, b2_ref, z_ref, sums_ref, acc_ref):
    m = jnp.concatenate([agg_ref[0], agg_ref[1]], axis=1)
    hid = jnp.dot(m, w1_ref[...], preferred_element_type=jnp.float32)
    hid = jnp.maximum(hid + b1_ref[...], 0.0)
    z = jnp.dot(hid, w2_ref[...], preferred_element_type=jnp.float32)
    z = z + b2_ref[...]
    z_ref[...] = z
    i = pl.program_id(0)

    @pl.when(i == 0)
    def _init():
        acc_ref[...] = jnp.zeros_like(acc_ref)

    acc_ref[0:1, :] += jnp.sum(z, axis=0, keepdims=True)
    acc_ref[1:2, :] += jnp.sum(z * z, axis=0, keepdims=True)

    @pl.when(i == pl.num_programs(0) - 1)
    def _fin():
        sums_ref[...] = acc_ref[...]


def _mlp_call(agg2, W1, b1, W2, b2):
    nb = N // _BN_ROWS
    return pl.pallas_call(
        _mlp_body,
        grid=(nb,),
        in_specs=[
            pl.BlockSpec((NC, _BN_ROWS, HALF), lambda i: (0, i, 0)),
            pl.BlockSpec((D, HID), lambda i: (0, 0)),
            pl.BlockSpec((1, HID), lambda i: (0, 0)),
            pl.BlockSpec((HID, D), lambda i: (0, 0)),
            pl.BlockSpec((1, D), lambda i: (0, 0)),
        ],
        out_specs=[
            pl.BlockSpec((_BN_ROWS, D), lambda i: (i, 0)),
            pl.BlockSpec((8, D), lambda i: (0, 0)),
        ],
        out_shape=[
            jax.ShapeDtypeStruct((N, D), jnp.float32),
            jax.ShapeDtypeStruct((8, D), jnp.float32),
        ],
        scratch_shapes=[pltpu.VMEM((8, D), jnp.float32)],
    )(agg2, W1, b1, W2, b2)


def _bn_body(z_ref, sums_ref, g_ref, b_ref, o_ref, *, relu):
    inv_n = 1.0 / N
    mu = sums_ref[0:1, :] * inv_n
    var = sums_ref[1:2, :] * inv_n - mu * mu
    scale = lax.rsqrt(var + 1e-5) * g_ref[...]
    y = (z_ref[...] - mu) * scale + b_ref[...]
    if relu:
        y = jnp.maximum(y, 0.0)
    o_ref[...] = y


def _bn_call(z, sums, gamma, beta, relu):
    nb = N // _BN_ROWS
    return pl.pallas_call(
        functools.partial(_bn_body, relu=relu),
        grid=(nb,),
        in_specs=[
            pl.BlockSpec((_BN_ROWS, D), lambda i: (i, 0)),
            pl.BlockSpec((8, D), lambda i: (0, 0)),
            pl.BlockSpec((1, D), lambda i: (0, 0)),
            pl.BlockSpec((1, D), lambda i: (0, 0)),
        ],
        out_specs=pl.BlockSpec((_BN_ROWS, D), lambda i: (i, 0)),
        out_shape=jax.ShapeDtypeStruct((N, D), jnp.float32),
    )(z, sums, gamma, beta)


def kernel(x, edge_index, batch, W1s, b1s, W2s, b2s, gammas, betas):
    del batch
    src = edge_index[0]
    dst = edge_index[1]
    src_p = jnp.concatenate([src, jnp.zeros((E_PAD - E,), jnp.int32)])
    dst_p = jnp.concatenate([dst, jnp.full((E_PAD - E,), N, jnp.int32)])
    dst3 = dst_p.reshape(NS, CH, K)

    h = x
    for l in range(L):
        h2 = h.reshape(NC * N, HALF)
        h3 = h.reshape(NS, RPT, D)
        agg2 = _sc_segsum(h3, h2, src_p, dst3).reshape(NC, N, HALF)
        z, sums = _mlp_call(agg2, W1s[l], b1s[l].reshape(1, HID),
                            W2s[l], b2s[l].reshape(1, D))
        h = _bn_call(z, sums, gammas[l].reshape(1, D),
                     betas[l].reshape(1, D), relu=(l < L - 1))
    return h
